# baseline (speedup 1.0000x reference)
import jax
import jax.numpy as jnp
from jax import lax
from jax.experimental import pallas as pl
from jax.experimental.pallas import tpu as pltpu

COORD_W = 5.0
OBJ_W = 2.0
NOOBJ_W = 0.5
CLS_W = 1.0


def _softplus_neg_abs(x):
    return jnp.log1p(jnp.exp(-jnp.abs(x)))


def _yolo_kernel(nt_ref, predT_ref, preds_ref, tgt_ref, out_ref):
    P = predT_ref.shape[2]
    M = tgt_ref.shape[1]

    predT = predT_ref[0]
    tgt = tgt_ref[0]
    nt = nt_ref[pl.program_id(0)]
    midx = lax.broadcasted_iota(jnp.int32, (M, 1), 0)
    maskcol = (midx < nt).astype(jnp.float32)

    px1 = predT[0:1, :]
    py1 = predT[1:2, :]
    px2 = predT[2:3, :]
    py2 = predT[3:4, :]
    xobj = predT[4:5, :]
    tx1 = tgt[:, 0:1]
    ty1 = tgt[:, 1:2]
    tx2 = tgt[:, 2:3]
    ty2 = tgt[:, 3:4]

    wp = px2 - px1
    hp = py2 - py1
    wt = tx2 - tx1
    ht = ty2 - ty1
    area1 = wp * hp
    area2 = wt * ht

    ltx = jnp.maximum(px1, tx1)
    rbx = jnp.minimum(px2, tx2)
    dxr = rbx - ltx
    cx = jnp.maximum(dxr, 0.0)
    lty = jnp.maximum(py1, ty1)
    rby = jnp.minimum(py2, ty2)
    dyr = rby - lty
    cy = jnp.maximum(dyr, 0.0)
    inter = cx * cy
    union = (area1 + area2) - inter
    cxi = jnp.maximum((wp + wt) - dxr, 0.0)
    cyi = jnp.maximum((hp + ht) - dyr, 0.0)
    areai = cxi * cyi
    q = (inter * areai + union * union) * pl.reciprocal(
        union * areai, approx=True)

    cmax = jnp.max(q, axis=1, keepdims=True)
    rows = lax.broadcasted_iota(jnp.int32, (M, P), 1)
    carg = jnp.min(jnp.where(q == cmax, rows, P), axis=1,
                   keepdims=True)
    onehot = (rows == carg).astype(jnp.float32)

    cand = lax.dot_general(onehot, preds_ref[0],
                           (((1,), (0,)), ((), ())),
                           preferred_element_type=jnp.float32)

    maskP = jnp.max(onehot * maskcol, axis=0, keepdims=True)

    sp_row = _softplus_neg_abs(xobj)
    bce0_row = jnp.maximum(xobj, 0.0) + sp_row
    bce1_row = bce0_row - xobj
    bce0_all = jnp.sum(bce0_row)
    obj_s = jnp.sum(maskP * bce1_row)
    noobj_s = bce0_all - jnp.sum(maskP * bce0_row)

    coord_s = jnp.sum((2.0 - cmax) * maskcol)

    clsg = cand[:, 5:]
    tgtc = tgt[:, 5:]
    clsbce = (jnp.maximum(clsg, 0.0) - clsg * tgtc
              + _softplus_neg_abs(clsg))
    cls_s = jnp.sum(jnp.sum(clsbce, axis=1, keepdims=True) * maskcol)
    nval_s = jnp.sum(maskcol)

    zero = jnp.zeros((1, 1), jnp.float32)
    row = jnp.concatenate(
        [coord_s.reshape(1, 1), obj_s.reshape(1, 1), noobj_s.reshape(1, 1),
         cls_s.reshape(1, 1), nval_s.reshape(1, 1), zero, zero, zero],
        axis=1)
    out_ref[...] = row.reshape(1, 1, 8)


def kernel(predictions, targets, num_targets):
    B, P, A = predictions.shape
    M = targets.shape[1]
    predT = jnp.transpose(predictions[..., :5], (0, 2, 1))

    grid_spec = pltpu.PrefetchScalarGridSpec(
        num_scalar_prefetch=1,
        grid=(B,),
        in_specs=[
            pl.BlockSpec((1, 5, P), lambda b, nt: (b, 0, 0)),
            pl.BlockSpec((1, P, A), lambda b, nt: (b, 0, 0)),
            pl.BlockSpec((1, M, A), lambda b, nt: (b, 0, 0)),
        ],
        out_specs=pl.BlockSpec((1, 1, 8), lambda b, nt: (b, 0, 0)),
    )
    out = pl.pallas_call(
        _yolo_kernel,
        grid_spec=grid_spec,
        out_shape=jax.ShapeDtypeStruct((B, 1, 8), jnp.float32),
        compiler_params=pltpu.CompilerParams(
            dimension_semantics=("arbitrary",)),
    )(num_targets, predT, predictions, targets)

    sums = jnp.sum(out[:, 0, :], axis=0)
    red_coord = sums[0] / B * COORD_W
    red_obj = sums[1] / B * OBJ_W
    red_noobj = sums[2] / B * NOOBJ_W
    red_cls = sums[3] / jnp.maximum(sums[4], 1.0) * CLS_W
    total = red_coord + red_obj + red_noobj + red_cls
    return (total, red_coord, red_obj, red_noobj, red_cls)


# speedup vs baseline: 1.0016x; 1.0016x over previous
import jax
import jax.numpy as jnp
from jax import lax
from jax.experimental import pallas as pl
from jax.experimental.pallas import tpu as pltpu

COORD_W = 5.0
OBJ_W = 2.0
NOOBJ_W = 0.5
CLS_W = 1.0


def _softplus_neg_abs(x):
    return jnp.log1p(jnp.exp(-jnp.abs(x)))


def _yolo_kernel(nt_ref, predT_ref, preds_ref, tgt_ref, out_ref):
    P = predT_ref.shape[2]
    M = tgt_ref.shape[1]

    predT = predT_ref[0]
    tgt = tgt_ref[0]
    nt = nt_ref[pl.program_id(0)]
    midx = lax.broadcasted_iota(jnp.int32, (M, 1), 0)
    maskcol = (midx < nt).astype(jnp.float32)

    px1 = predT[0:1, :]
    py1 = predT[1:2, :]
    px2 = predT[2:3, :]
    py2 = predT[3:4, :]
    xobj = predT[4:5, :]
    tx1 = tgt[:, 0:1]
    ty1 = tgt[:, 1:2]
    tx2 = tgt[:, 2:3]
    ty2 = tgt[:, 3:4]

    wp = px2 - px1
    hp = py2 - py1
    wt = tx2 - tx1
    ht = ty2 - ty1
    area1 = wp * hp
    area2 = wt * ht

    ltx = jnp.maximum(px1, tx1)
    rbx = jnp.minimum(px2, tx2)
    dxr = rbx - ltx
    cx = jnp.maximum(dxr, 0.0)
    lty = jnp.maximum(py1, ty1)
    rby = jnp.minimum(py2, ty2)
    dyr = rby - lty
    cy = jnp.maximum(dyr, 0.0)
    inter = cx * cy
    union = (area1 + area2) - inter
    cxi = jnp.maximum((wp + wt) - dxr, 0.0)
    cyi = jnp.maximum((hp + ht) - dyr, 0.0)
    areai = cxi * cyi
    q = (inter * areai + union * union) / (union * areai)

    cmax = jnp.max(q, axis=1, keepdims=True)
    rows = lax.broadcasted_iota(jnp.int32, (M, P), 1)
    carg = jnp.min(jnp.where(q == cmax, rows, P), axis=1,
                   keepdims=True)
    onehot = (rows == carg).astype(jnp.float32)

    cand = lax.dot_general(onehot, preds_ref[0],
                           (((1,), (0,)), ((), ())),
                           preferred_element_type=jnp.float32)

    maskP = jnp.max(onehot * maskcol, axis=0, keepdims=True)

    sp_row = _softplus_neg_abs(xobj)
    bce0_row = jnp.maximum(xobj, 0.0) + sp_row
    bce1_row = bce0_row - xobj
    bce0_all = jnp.sum(bce0_row)
    obj_s = jnp.sum(maskP * bce1_row)
    noobj_s = bce0_all - jnp.sum(maskP * bce0_row)

    coord_s = jnp.sum((2.0 - cmax) * maskcol)

    clsg = cand[:, 5:]
    tgtc = tgt[:, 5:]
    clsbce = (jnp.maximum(clsg, 0.0) - clsg * tgtc
              + _softplus_neg_abs(clsg))
    cls_s = jnp.sum(jnp.sum(clsbce, axis=1, keepdims=True) * maskcol)
    nval_s = jnp.sum(maskcol)

    zero = jnp.zeros((1, 1), jnp.float32)
    row = jnp.concatenate(
        [coord_s.reshape(1, 1), obj_s.reshape(1, 1), noobj_s.reshape(1, 1),
         cls_s.reshape(1, 1), nval_s.reshape(1, 1), zero, zero, zero],
        axis=1)
    out_ref[...] = row.reshape(1, 1, 8)


def kernel(predictions, targets, num_targets):
    B, P, A = predictions.shape
    M = targets.shape[1]
    predT = jnp.transpose(predictions[..., :5], (0, 2, 1))

    grid_spec = pltpu.PrefetchScalarGridSpec(
        num_scalar_prefetch=1,
        grid=(B,),
        in_specs=[
            pl.BlockSpec((1, 5, P), lambda b, nt: (b, 0, 0)),
            pl.BlockSpec((1, P, A), lambda b, nt: (b, 0, 0)),
            pl.BlockSpec((1, M, A), lambda b, nt: (b, 0, 0)),
        ],
        out_specs=pl.BlockSpec((1, 1, 8), lambda b, nt: (b, 0, 0)),
    )
    out = pl.pallas_call(
        _yolo_kernel,
        grid_spec=grid_spec,
        out_shape=jax.ShapeDtypeStruct((B, 1, 8), jnp.float32),
        compiler_params=pltpu.CompilerParams(
            dimension_semantics=("arbitrary",)),
    )(num_targets, predT, predictions, targets)

    sums = jnp.sum(out[:, 0, :], axis=0)
    red_coord = sums[0] / B * COORD_W
    red_obj = sums[1] / B * OBJ_W
    red_noobj = sums[2] / B * NOOBJ_W
    red_cls = sums[3] / jnp.maximum(sums[4], 1.0) * CLS_W
    total = red_coord + red_obj + red_noobj + red_cls
    return (total, red_coord, red_obj, red_noobj, red_cls)
